# baseline (speedup 1.0000x reference)
import functools

import jax
import jax.numpy as jnp
from jax import lax
from jax.experimental import pallas as pl
from jax.experimental.pallas import tpu as pltpu
from jax.experimental.pallas import tpu_sc as plsc

G = 128
N = 200000
L = 16
W = 400
NCH = N // W
NC = 2
NS = 16
NW = NC * NS
KMAX = (NCH + NW - 1) // NW

_MESH = plsc.VectorSubcoreMesh(core_axis_name="c", subcore_axis_name="s")
_PARAMS = pltpu.CompilerParams(use_tc_tiling_on_sc=False,
                               needs_layout_passes=False)


def _widx():
    return lax.axis_index("s") * NC + lax.axis_index("c")


@functools.partial(
    pl.kernel,
    out_type=[
        jax.ShapeDtypeStruct((N,), jnp.float32),
        jax.ShapeDtypeStruct((N,), jnp.int32),
        jax.ShapeDtypeStruct((NW, G, L), jnp.float32),
    ],
    mesh=_MESH,
    compiler_params=_PARAMS,
    scratch_types=[
        pltpu.VMEM((G, W), jnp.float32),
        pltpu.VMEM((G, L), jnp.float32),
        pltpu.VMEM((W,), jnp.float32),
        pltpu.VMEM((W,), jnp.int32),
    ],
)
def _k1(ov_hbm, maxov_hbm, pre_hbm, part_hbm, buf, racc, cmbuf, prebuf):
    w = _widx()

    def init_racc(r, _):
        racc[r] = jnp.full((L,), -1.0, jnp.float32)
        return 0
    lax.fori_loop(0, G, init_racc, 0)

    def chunk_body(k, _):
        c = k * NW + w

        @pl.when(c < NCH)
        def _():
            pltpu.sync_copy(ov_hbm.at[:, pl.ds(c * W, W)], buf)

            def g_body(g, _):
                def r_body(r, carry):
                    cm, ai = carry
                    v = buf[r, pl.ds(g * L, L)]
                    m = v > cm
                    cm = jnp.where(m, v, cm)
                    ai = jnp.where(m, jnp.zeros((L,), jnp.int32) + r, ai)
                    racc[r] = jnp.maximum(racc[r], v)
                    return cm, ai

                cm0 = jnp.full((L,), -1.0, jnp.float32)
                ai0 = jnp.zeros((L,), jnp.int32)
                cm, ai = lax.fori_loop(0, G, r_body, (cm0, ai0))
                neg = cm < 0.4
                pos = (cm > 0.5) & (cm < 0.8)
                a = jnp.where(neg, jnp.zeros((L,), jnp.int32),
                              jnp.full((L,), -1, jnp.int32))
                a = jnp.where(pos, ai + 1, a)
                cmbuf[pl.ds(g * L, L)] = cm
                prebuf[pl.ds(g * L, L)] = a
                return 0

            lax.fori_loop(0, W // L, g_body, 0)
            pltpu.sync_copy(cmbuf, maxov_hbm.at[pl.ds(c * W, W)])
            pltpu.sync_copy(prebuf, pre_hbm.at[pl.ds(c * W, W)])
        return 0

    lax.fori_loop(0, KMAX, chunk_body, 0)
    pltpu.sync_copy(racc, part_hbm.at[w])


@functools.partial(
    pl.kernel,
    out_type=jax.ShapeDtypeStruct((N,), jnp.int32),
    mesh=_MESH,
    compiler_params=_PARAMS,
    scratch_types=[
        pltpu.VMEM((G, W), jnp.float32),
        pltpu.VMEM((NW, G, L), jnp.float32),
        pltpu.VMEM((G, L), jnp.float32),
        pltpu.VMEM((W,), jnp.int32),
        pltpu.VMEM((W,), jnp.int32),
    ],
)
def _k2(ov_hbm, pre_hbm, part_hbm, asg_hbm, buf, pall, gtb, prebuf, outbuf):
    w = _widx()
    pltpu.sync_copy(part_hbm, pall)

    def gt_body(r, _):
        def w_body(w2, acc):
            return jnp.maximum(acc, pall[w2, r])
        acc = lax.fori_loop(0, NW, w_body, jnp.full((L,), -1.0, jnp.float32))
        gtb[r] = jnp.zeros((L,), jnp.float32) + jnp.max(acc)
        return 0
    lax.fori_loop(0, G, gt_body, 0)

    def chunk_body(k, _):
        c = k * NW + w

        @pl.when(c < NCH)
        def _():
            pltpu.sync_copy(ov_hbm.at[:, pl.ds(c * W, W)], buf)
            pltpu.sync_copy(pre_hbm.at[pl.ds(c * W, W)], prebuf)

            def g_body(g, _):
                def r_body(r, best):
                    v = buf[r, pl.ds(g * L, L)]
                    m = v == gtb[r]
                    return jnp.where(m, jnp.zeros((L,), jnp.int32) + (r + 1),
                                     best)

                best = lax.fori_loop(0, G, r_body, jnp.zeros((L,), jnp.int32))
                p = prebuf[pl.ds(g * L, L)]
                outbuf[pl.ds(g * L, L)] = jnp.where(best > 0, best, p)
                return 0

            lax.fori_loop(0, W // L, g_body, 0)
            pltpu.sync_copy(outbuf, asg_hbm.at[pl.ds(c * W, W)])
        return 0

    lax.fori_loop(0, KMAX, chunk_body, 0)


def kernel(overlaps):
    maxov, pre, part = _k1(overlaps)
    assigned = _k2(overlaps, pre, part)
    return assigned, maxov


# speedup vs baseline: 2.0622x; 2.0622x over previous
import functools

import jax
import jax.numpy as jnp
from jax import lax
from jax.experimental import pallas as pl
from jax.experimental.pallas import tpu as pltpu
from jax.experimental.pallas import tpu_sc as plsc

G = 128
N = 200000
L = 16
W = 400
GPC = W // L
NCH = N // W
NC = 2
NS = 16
NW = NC * NS
KMAX = (NCH + NW - 1) // NW
RB = 8

_MESH = plsc.VectorSubcoreMesh(core_axis_name="c", subcore_axis_name="s")
_PARAMS = pltpu.CompilerParams(use_tc_tiling_on_sc=False,
                               needs_layout_passes=False)


def _widx():
    return lax.axis_index("s") * NC + lax.axis_index("c")


@functools.partial(
    pl.kernel,
    out_type=[
        jax.ShapeDtypeStruct((N,), jnp.float32),
        jax.ShapeDtypeStruct((N,), jnp.int32),
        jax.ShapeDtypeStruct((NW, G, L), jnp.float32),
    ],
    mesh=_MESH,
    compiler_params=_PARAMS,
    scratch_types=[
        pltpu.VMEM((G, W), jnp.float32),
        pltpu.VMEM((G, W), jnp.float32),
        pltpu.VMEM((G, L), jnp.float32),
        pltpu.VMEM((W,), jnp.float32),
        pltpu.VMEM((W,), jnp.int32),
        pltpu.VMEM((W,), jnp.int32),
        pltpu.SemaphoreType.DMA,
        pltpu.SemaphoreType.DMA,
    ],
)
def _k1(ov_hbm, maxov_hbm, pre_hbm, part_hbm,
        buf0, buf1, racc, cmbuf, aibuf, prebuf, sem0, sem1):
    w = _widx()

    def init_racc(r, _):
        racc[r] = jnp.full((L,), -1.0, jnp.float32)
        return 0
    lax.fori_loop(0, G, init_racc, 0)

    def chunk_of(k):
        return k * NW + w

    def start(k, buf, sem):
        c = chunk_of(k)

        @pl.when(c < NCH)
        def _():
            pltpu.make_async_copy(
                ov_hbm.at[:, pl.ds(c * W, W)], buf, sem).start()

    def compute(k, buf, sem):
        c = chunk_of(k)

        @pl.when(c < NCH)
        def _():
            pltpu.make_async_copy(
                ov_hbm.at[:, pl.ds(c * W, W)], buf, sem).wait()

            def init_g(g, _):
                cmbuf[pl.ds(g * L, L)] = jnp.full((L,), -1.0, jnp.float32)
                aibuf[pl.ds(g * L, L)] = jnp.zeros((L,), jnp.int32)
                return 0
            lax.fori_loop(0, GPC, init_g, 0)

            def rb_body(rb, _):
                r0 = rb * RB

                def g_body(g, raccs):
                    gl = g * L
                    cm = cmbuf[pl.ds(gl, L)]
                    ai = aibuf[pl.ds(gl, L)]
                    out = []
                    for i in range(RB):
                        v = buf[r0 + i, pl.ds(gl, L)]
                        m = v > cm
                        cm = jnp.where(m, v, cm)
                        ai = jnp.where(
                            m, jnp.zeros((L,), jnp.int32) + (r0 + i), ai)
                        out.append(jnp.maximum(raccs[i], v))
                    cmbuf[pl.ds(gl, L)] = cm
                    aibuf[pl.ds(gl, L)] = ai
                    return tuple(out)

                init = tuple(racc[r0 + i] for i in range(RB))
                fin = lax.fori_loop(0, GPC, g_body, init)
                for i in range(RB):
                    racc[r0 + i] = fin[i]
                return 0

            lax.fori_loop(0, G // RB, rb_body, 0)

            def pre_body(g, _):
                gl = g * L
                cm = cmbuf[pl.ds(gl, L)]
                ai = aibuf[pl.ds(gl, L)]
                neg = cm < 0.4
                pos = (cm > 0.5) & (cm < 0.8)
                a = jnp.where(neg, jnp.zeros((L,), jnp.int32),
                              jnp.full((L,), -1, jnp.int32))
                a = jnp.where(pos, ai + 1, a)
                prebuf[pl.ds(gl, L)] = a
                return 0
            lax.fori_loop(0, GPC, pre_body, 0)

            pltpu.sync_copy(cmbuf, maxov_hbm.at[pl.ds(c * W, W)])
            pltpu.sync_copy(prebuf, pre_hbm.at[pl.ds(c * W, W)])

    start(0, buf0, sem0)

    def outer(kk, _):
        k0 = 2 * kk
        start(k0 + 1, buf1, sem1)
        compute(k0, buf0, sem0)
        start(k0 + 2, buf0, sem0)
        compute(k0 + 1, buf1, sem1)
        return 0
    lax.fori_loop(0, KMAX // 2, outer, 0)

    pltpu.sync_copy(racc, part_hbm.at[w])


@functools.partial(
    pl.kernel,
    out_type=jax.ShapeDtypeStruct((N,), jnp.int32),
    mesh=_MESH,
    compiler_params=_PARAMS,
    scratch_types=[
        pltpu.VMEM((NW, G, L), jnp.float32),
        pltpu.VMEM((G, L), jnp.float32),
        pltpu.VMEM((W,), jnp.int32),
        pltpu.VMEM((W,), jnp.float32),
        pltpu.SMEM((G,), jnp.float32),
        pltpu.SMEM((G,), jnp.int32),
    ],
)
def _k2(ov_hbm, pre_hbm, part_hbm, asg_hbm,
        pall, gtb, outbuf, rowbuf, gts, rows):
    w = _widx()
    pltpu.sync_copy(part_hbm, pall)

    def gt_body(r, _):
        def w_body(w2, acc):
            return jnp.maximum(acc, pall[w2, r])
        acc = lax.fori_loop(0, NW, w_body, jnp.full((L,), -1.0, jnp.float32))
        s = jnp.max(acc)
        gtb[r] = jnp.zeros((L,), jnp.float32) + s
        gts[r] = s
        return 0
    lax.fori_loop(0, G, gt_body, 0)

    def cand_body(r, cnt):
        tie = jnp.max(pall[w, r]) == gts[r]

        @pl.when(tie)
        def _():
            rows[cnt] = r
        return jnp.where(tie, cnt + 1, cnt)
    ncand = lax.fori_loop(0, G, cand_body, jnp.int32(0))

    def chunk_body(k, _):
        c = k * NW + w

        @pl.when(c < NCH)
        def _():
            pltpu.sync_copy(pre_hbm.at[pl.ds(c * W, W)], outbuf)

            def row_body(i, _):
                r = rows[i]
                pltpu.sync_copy(ov_hbm.at[r, pl.ds(c * W, W)], rowbuf)
                rp1 = jnp.zeros((L,), jnp.int32) + (r + 1)

                def g_body(g, _):
                    gl = g * L
                    v = rowbuf[pl.ds(gl, L)]
                    m = v == gtb[r]
                    outbuf[pl.ds(gl, L)] = jnp.where(m, rp1,
                                                     outbuf[pl.ds(gl, L)])
                    return 0
                lax.fori_loop(0, GPC, g_body, 0)
                return 0
            lax.fori_loop(0, ncand, row_body, 0)

            pltpu.sync_copy(outbuf, asg_hbm.at[pl.ds(c * W, W)])
        return 0

    lax.fori_loop(0, KMAX, chunk_body, 0)


def kernel(overlaps):
    maxov, pre, part = _k1(overlaps)
    assigned = _k2(overlaps, pre, part)
    return assigned, maxov
